# baseline (speedup 1.0000x reference)
import functools

import jax
import jax.numpy as jnp
import numpy as np
from jax import lax
from jax.experimental import pallas as pl
from jax.experimental.pallas import tpu as pltpu
from jax.experimental.pallas import tpu_sc as plsc

NF = 26
ED = 16
NT = 28
ROWI = NT * ED // 2
TOT = 104000
B = 4096
NW = 32
GROUP = 2
GR = GROUP * NF
NG = B // GROUP
GPW = NG // NW
NIT = GPW // 2

_OFFSETS = np.arange(NF, dtype=np.int32) * 4000
_HI = np.int32(-65536)


def _sc_body(sidx_hbm, bias_hbm, wt_hbm, out_hbm,
             idx_a, idx_b, gbuf_a, gbuf_b, res_v, bias_v,
             sem_a, sem_b, isem_a, isem_b):
    wid = lax.axis_index("s") * 2 + lax.axis_index("c")
    pltpu.sync_copy(bias_hbm, bias_v)
    bvec = bias_v[...]
    lanes = jnp.arange(16, dtype=jnp.int32)
    mask8 = lanes < 8
    mask8f = jnp.where(mask8, 1.0, 0.0)

    def lperm(val, perm):
        return lax.gather(
            val, perm[:, None],
            dimension_numbers=lax.GatherDimensionNumbers(
                offset_dims=(), collapsed_slice_dims=(0,),
                start_index_map=(0,)),
            slice_sizes=(1,),
            mode=lax.GatherScatterMode.PROMISE_IN_BOUNDS)

    def ev(v):
        return lax.bitcast_convert_type(v << 16, jnp.float32)

    def od(v):
        return lax.bitcast_convert_type(v & _HI, jnp.float32)

    def compute_sample(gbuf, sbase):
        accs = [jnp.zeros((ED,), jnp.float32) for _ in range(4)]
        n = 0
        for i in range(NF):
            js = list(range(i + 1, NF))
            k = 0
            while k + 1 < len(js):
                j = js[k]
                va = gbuf[sbase + i, pl.ds(j * 8, 16)]
                b1 = gbuf[sbase + j, pl.ds(i * 8, 16)]
                b2 = gbuf[sbase + j + 1, pl.ds(i * 8, 16)]
                vb = jnp.where(mask8, b1, lperm(b2, lanes ^ 8))
                accs[n & 3] = accs[n & 3] + (ev(va) * ev(vb)
                                             + od(va) * od(vb))
                n += 1
                k += 2
            if k < len(js):
                j = js[k]
                va = gbuf[sbase + i, pl.ds(j * 8, 16)]
                vb = gbuf[sbase + j, pl.ds(i * 8, 16)]
                accs[n & 3] = accs[n & 3] + mask8f * (ev(va) * ev(vb)
                                                      + od(va) * od(vb))
                n += 1
        lacc = jnp.zeros((ED,), jnp.float32)
        for j in range(NF):
            vl = gbuf[sbase + j, pl.ds(NF * 8, 16)]
            lacc = lacc + (ev(vl) + od(vl))
        return (accs[0] + accs[1]) + (accs[2] + accs[3]) + lacc

    g0 = wid * GPW
    pltpu.sync_copy(sidx_hbm.at[g0], idx_a)
    pltpu.async_copy(wt_hbm.at[idx_a], gbuf_a, sem_a)
    pltpu.async_copy(sidx_hbm.at[g0 + 1], idx_b, isem_b)

    def it_body(t, resvec):
        for gi, (gbuf, gbuf_o, idx_o, sem_o, idx_p, isem_p, sem_w,
                 isem_w) in enumerate((
                (gbuf_a, gbuf_b, idx_b, sem_b, idx_a, isem_a, sem_a, isem_b),
                (gbuf_b, gbuf_a, idx_a, sem_a, idx_b, isem_b, sem_b, isem_a))):
            pltpu.make_async_copy(sidx_hbm.at[g0], idx_o, isem_w).wait()
            pltpu.async_copy(wt_hbm.at[idx_o], gbuf_o, sem_o)
            pltpu.make_async_copy(wt_hbm.at[pl.ds(0, GR)], gbuf, sem_w).wait()
            pltpu.async_copy(sidx_hbm.at[g0 + 2 * t + gi + 2], idx_p, isem_p)
            for s in range(GROUP):
                tot = compute_sample(gbuf, s * NF)
                for sh in (8, 4, 2, 1):
                    tot = tot + lperm(tot, lanes ^ sh)
                lane_val = (4 * t + 2 * gi + s) & 15
                resvec = resvec + jnp.where(lanes == lane_val, tot, 0.0)
        sig = 1.0 / (1.0 + jnp.exp(-(resvec + bvec)))
        res_v[pl.ds((t // 4) * 16, 16)] = sig
        return jnp.where((t & 3) == 3, jnp.zeros((16,), jnp.float32), resvec)

    lax.fori_loop(0, NIT, it_body, jnp.zeros((16,), jnp.float32))
    pltpu.make_async_copy(wt_hbm.at[pl.ds(0, GR)], gbuf_a, sem_a).wait()
    pltpu.make_async_copy(sidx_hbm.at[g0], idx_b, isem_b).wait()
    pltpu.sync_copy(res_v, out_hbm.at[pl.ds(wid * (B // NW), B // NW)])


@functools.partial(
    pl.kernel,
    mesh=plsc.VectorSubcoreMesh(core_axis_name="c", subcore_axis_name="s"),
    out_type=jax.ShapeDtypeStruct((B,), jnp.float32),
    compiler_params=pltpu.CompilerParams(use_tc_tiling_on_sc=False),
    scratch_types=[
        pltpu.VMEM((GR,), jnp.int32),
        pltpu.VMEM((GR,), jnp.int32),
        pltpu.VMEM((GR, ROWI), jnp.int32),
        pltpu.VMEM((GR, ROWI), jnp.int32),
        pltpu.VMEM((B // NW,), jnp.float32),
        pltpu.VMEM((16,), jnp.float32),
        pltpu.SemaphoreType.DMA,
        pltpu.SemaphoreType.DMA,
        pltpu.SemaphoreType.DMA,
        pltpu.SemaphoreType.DMA,
    ],
)
def _ffm_sc(sidx_hbm, bias_hbm, wt_hbm, out_hbm,
            idx_a, idx_b, gbuf_a, gbuf_b, res_v, bias_v,
            sem_a, sem_b, isem_a, isem_b):
    _sc_body(sidx_hbm, bias_hbm, wt_hbm, out_hbm,
             idx_a, idx_b, gbuf_a, gbuf_b, res_v, bias_v,
             sem_a, sem_b, isem_a, isem_b)


def kernel(x, linear_w, bias, ffm_w):
    idx = (x.astype(jnp.int32) + jnp.asarray(_OFFSETS)[None, :])
    sidx = jnp.pad(idx.reshape(NG, GR), ((0, 4), (0, 0)))
    bias16 = jnp.broadcast_to(bias.astype(jnp.float32), (16,))
    linp = jnp.pad(linear_w.astype(jnp.float32), ((0, 0), (0, ED - 1)))
    ffm_bf = lax.optimization_barrier(ffm_w.astype(jnp.bfloat16))
    lin_bf = linp.astype(jnp.bfloat16)
    wtf = jnp.concatenate(
        [ffm_bf.transpose(1, 0, 2), lin_bf[:, None, :],
         jnp.zeros((TOT, 1, ED), jnp.bfloat16)], axis=1
    ).reshape(TOT, ROWI, 2)
    wti = lax.bitcast_convert_type(wtf, jnp.int32)
    return _ffm_sc(sidx, bias16, wti)


# speedup vs baseline: 1.0018x; 1.0018x over previous
import functools

import jax
import jax.numpy as jnp
import numpy as np
from jax import lax
from jax.experimental import pallas as pl
from jax.experimental.pallas import tpu as pltpu
from jax.experimental.pallas import tpu_sc as plsc

NF = 26
ED = 16
NT = 28
ROWI = NT * ED // 2
TOT = 104000
B = 4096
NW = 32
GROUP = 2
GR = GROUP * NF
NG = B // GROUP
GPW = NG // NW
NIT = GPW // 2

_OFFSETS = np.arange(NF, dtype=np.int32) * 4000
_HI = np.int32(-65536)


def _sc_body(sidx_hbm, bias_hbm, wt_hbm, out_hbm,
             idx_a, idx_b, gbuf_a, gbuf_b, res_v, bias_v,
             sem_a, sem_b, isem_a, isem_b):
    wid = lax.axis_index("s") * 2 + lax.axis_index("c")
    pltpu.sync_copy(bias_hbm, bias_v)
    bvec = bias_v[...]
    lanes = jnp.arange(16, dtype=jnp.int32)
    mask8 = lanes < 8
    mask8f = jnp.where(mask8, 1.0, 0.0)

    def lperm(val, perm):
        return lax.gather(
            val, perm[:, None],
            dimension_numbers=lax.GatherDimensionNumbers(
                offset_dims=(), collapsed_slice_dims=(0,),
                start_index_map=(0,)),
            slice_sizes=(1,),
            mode=lax.GatherScatterMode.PROMISE_IN_BOUNDS)

    def ev(v):
        return lax.bitcast_convert_type(v << 16, jnp.float32)

    def od(v):
        return lax.bitcast_convert_type(v & _HI, jnp.float32)

    def compute_sample(gbuf, sbase):
        accs = [jnp.zeros((ED,), jnp.float32) for _ in range(4)]
        n = 0
        for i in range(NF):
            js = list(range(i + 1, NF))
            k = 0
            while k + 1 < len(js):
                j = js[k]
                va = gbuf[sbase + i, pl.ds(j * 8, 16)]
                b1 = gbuf[sbase + j, pl.ds(i * 8, 16)]
                b2 = gbuf[sbase + j + 1, pl.ds(i * 8, 16)]
                vb = jnp.where(mask8, b1, lperm(b2, lanes ^ 8))
                accs[n & 3] = accs[n & 3] + (ev(va) * ev(vb)
                                             + od(va) * od(vb))
                n += 1
                k += 2
            if k < len(js):
                j = js[k]
                va = gbuf[sbase + i, pl.ds(j * 8, 16)]
                vb = gbuf[sbase + j, pl.ds(i * 8, 16)]
                accs[n & 3] = accs[n & 3] + mask8f * (ev(va) * ev(vb)
                                                      + od(va) * od(vb))
                n += 1
        lacc = jnp.zeros((ED,), jnp.float32)
        for j in range(NF):
            vl = gbuf[sbase + j, pl.ds(NF * 8, 16)]
            lacc = lacc + (ev(vl) + od(vl))
        return (accs[0] + accs[1]) + (accs[2] + accs[3]) + lacc

    g0 = wid * GPW
    pltpu.sync_copy(sidx_hbm.at[g0], idx_a)
    pltpu.async_copy(wt_hbm.at[idx_a], gbuf_a, sem_a)
    pltpu.async_copy(sidx_hbm.at[g0 + 1], idx_b, isem_b)

    def it_body(t, resvec):
        for gi, (gbuf, gbuf_o, idx_o, sem_o, idx_p, isem_p, sem_w,
                 isem_w) in enumerate((
                (gbuf_a, gbuf_b, idx_b, sem_b, idx_a, isem_a, sem_a, isem_b),
                (gbuf_b, gbuf_a, idx_a, sem_a, idx_b, isem_b, sem_b, isem_a))):
            pltpu.make_async_copy(sidx_hbm.at[g0], idx_o, isem_w).wait()
            pltpu.async_copy(wt_hbm.at[idx_o], gbuf_o, sem_o)
            pltpu.make_async_copy(wt_hbm.at[pl.ds(0, GR)], gbuf, sem_w).wait()
            pltpu.async_copy(sidx_hbm.at[g0 + 2 * t + gi + 2], idx_p, isem_p)
            for s in range(GROUP):
                tot = compute_sample(gbuf, s * NF)
                for sh in (8, 4, 2, 1):
                    tot = tot + lperm(tot, lanes ^ sh)
                lane_val = (4 * t + 2 * gi + s) & 15
                resvec = resvec + jnp.where(lanes == lane_val, tot, 0.0)
        sig = 1.0 / (1.0 + jnp.exp(-(resvec + bvec)))
        res_v[pl.ds((t // 4) * 16, 16)] = sig
        return jnp.where((t & 3) == 3, jnp.zeros((16,), jnp.float32), resvec)

    lax.fori_loop(0, NIT, it_body, jnp.zeros((16,), jnp.float32))
    pltpu.make_async_copy(wt_hbm.at[pl.ds(0, GR)], gbuf_a, sem_a).wait()
    pltpu.make_async_copy(sidx_hbm.at[g0], idx_b, isem_b).wait()
    pltpu.sync_copy(res_v, out_hbm.at[pl.ds(wid * (B // NW), B // NW)])


@functools.partial(
    pl.kernel,
    mesh=plsc.VectorSubcoreMesh(core_axis_name="c", subcore_axis_name="s"),
    out_type=jax.ShapeDtypeStruct((B,), jnp.float32),
    compiler_params=pltpu.CompilerParams(use_tc_tiling_on_sc=False),
    scratch_types=[
        pltpu.VMEM((GR,), jnp.int32),
        pltpu.VMEM((GR,), jnp.int32),
        pltpu.VMEM((GR, ROWI), jnp.int32),
        pltpu.VMEM((GR, ROWI), jnp.int32),
        pltpu.VMEM((B // NW,), jnp.float32),
        pltpu.VMEM((16,), jnp.float32),
        pltpu.SemaphoreType.DMA,
        pltpu.SemaphoreType.DMA,
        pltpu.SemaphoreType.DMA,
        pltpu.SemaphoreType.DMA,
    ],
)
def _ffm_sc(sidx_hbm, bias_hbm, wt_hbm, out_hbm,
            idx_a, idx_b, gbuf_a, gbuf_b, res_v, bias_v,
            sem_a, sem_b, isem_a, isem_b):
    _sc_body(sidx_hbm, bias_hbm, wt_hbm, out_hbm,
             idx_a, idx_b, gbuf_a, gbuf_b, res_v, bias_v,
             sem_a, sem_b, isem_a, isem_b)


def kernel(x, linear_w, bias, ffm_w):
    idx = (x.astype(jnp.int32) + jnp.asarray(_OFFSETS)[None, :])
    sidx = jnp.pad(idx.reshape(NG, GR), ((0, 4), (0, 0)))
    bias16 = jnp.broadcast_to(bias.astype(jnp.float32), (16,))
    linp = jnp.pad(linear_w.astype(jnp.float32), ((0, 0), (0, ED - 1)))
    wtf = jnp.concatenate(
        [ffm_w.transpose(1, 0, 2), linp[:, None, :],
         jnp.zeros((TOT, 1, ED), jnp.float32)], axis=1
    ).astype(jnp.bfloat16).reshape(TOT, ROWI, 2)
    wti = lax.bitcast_convert_type(wtf, jnp.int32)
    return _ffm_sc(sidx, bias16, wti)


# speedup vs baseline: 1.0248x; 1.0230x over previous
import functools

import jax
import jax.numpy as jnp
import numpy as np
from jax import lax
from jax.experimental import pallas as pl
from jax.experimental.pallas import tpu as pltpu
from jax.experimental.pallas import tpu_sc as plsc

NF = 26
ED = 16
NT = 28
ROWI = NT * ED // 2
TOT = 104000
B = 4096
NW = 32
GROUP = 2
GR = GROUP * NF
NG = B // GROUP
GPW = NG // NW
NIT = GPW // 2

_OFFSETS = np.arange(NF, dtype=np.int32) * 4000
_HI = np.int32(-65536)


def _sc_body(sidx_hbm, bias_hbm, wt_hbm, out_hbm,
             idx_a, idx_b, gbuf_a, gbuf_b, res_v, bias_v,
             sem_a, sem_b, isem_a, isem_b):
    wid = lax.axis_index("s") * 2 + lax.axis_index("c")
    pltpu.sync_copy(bias_hbm, bias_v)
    bvec = bias_v[...]
    lanes = jnp.arange(16, dtype=jnp.int32)
    mask8 = lanes < 8
    mask8f = jnp.where(mask8, 1.0, 0.0)

    def lperm(val, perm):
        return lax.gather(
            val, perm[:, None],
            dimension_numbers=lax.GatherDimensionNumbers(
                offset_dims=(), collapsed_slice_dims=(0,),
                start_index_map=(0,)),
            slice_sizes=(1,),
            mode=lax.GatherScatterMode.PROMISE_IN_BOUNDS)

    def ev(v):
        return lax.bitcast_convert_type(v << 16, jnp.float32)

    def od(v):
        return lax.bitcast_convert_type(v & _HI, jnp.float32)

    def compute_sample(gbuf, sbase):
        accs = [jnp.zeros((ED,), jnp.float32) for _ in range(4)]
        n = 0
        for i in range(NF):
            js = list(range(i + 1, NF))
            k = 0
            while k + 1 < len(js):
                j = js[k]
                va = gbuf[sbase + i, pl.ds(j * 8, 16)]
                b1 = gbuf[sbase + j, pl.ds(i * 8, 16)]
                b2 = gbuf[sbase + j + 1, pl.ds(i * 8, 16)]
                vb = jnp.where(mask8, b1, lperm(b2, lanes ^ 8))
                accs[n & 3] = accs[n & 3] + (ev(va) * ev(vb)
                                             + od(va) * od(vb))
                n += 1
                k += 2
            if k < len(js):
                j = js[k]
                va = gbuf[sbase + i, pl.ds(j * 8, 16)]
                vb = gbuf[sbase + j, pl.ds(i * 8, 16)]
                accs[n & 3] = accs[n & 3] + mask8f * (ev(va) * ev(vb)
                                                      + od(va) * od(vb))
                n += 1
        lacc = jnp.zeros((ED,), jnp.float32)
        for j in range(NF):
            vl = gbuf[sbase + j, pl.ds(NF * 8, 16)]
            lacc = lacc + (ev(vl) + od(vl))
        return (accs[0] + accs[1]) + (accs[2] + accs[3]) + lacc

    g0 = wid * GPW
    pltpu.sync_copy(sidx_hbm.at[g0], idx_a)
    pltpu.async_copy(wt_hbm.at[idx_a], gbuf_a, sem_a)
    pltpu.async_copy(sidx_hbm.at[g0 + 1], idx_b, isem_b)

    def it_body(t, resvec):
        for gi, (gbuf, gbuf_o, idx_o, sem_o, idx_p, isem_p, sem_w,
                 isem_w) in enumerate((
                (gbuf_a, gbuf_b, idx_b, sem_b, idx_a, isem_a, sem_a, isem_b),
                (gbuf_b, gbuf_a, idx_a, sem_a, idx_b, isem_b, sem_b, isem_a))):
            pltpu.make_async_copy(sidx_hbm.at[g0], idx_o, isem_w).wait()
            pltpu.async_copy(wt_hbm.at[idx_o], gbuf_o, sem_o)
            pltpu.make_async_copy(wt_hbm.at[pl.ds(0, GR)], gbuf, sem_w).wait()
            pltpu.async_copy(sidx_hbm.at[g0 + 2 * t + gi + 2], idx_p, isem_p)
            for s in range(GROUP):
                tot = compute_sample(gbuf, s * NF)
                for sh in (8, 4, 2, 1):
                    tot = tot + lperm(tot, lanes ^ sh)
                lane_val = (4 * t + 2 * gi + s) & 15
                resvec = resvec + jnp.where(lanes == lane_val, tot, 0.0)
        sig = 1.0 / (1.0 + jnp.exp(-(resvec + bvec)))
        res_v[pl.ds((t // 4) * 16, 16)] = sig
        return jnp.where((t & 3) == 3, jnp.zeros((16,), jnp.float32), resvec)

    lax.fori_loop(0, NIT, it_body, jnp.zeros((16,), jnp.float32))
    pltpu.make_async_copy(wt_hbm.at[pl.ds(0, GR)], gbuf_a, sem_a).wait()
    pltpu.make_async_copy(sidx_hbm.at[g0], idx_b, isem_b).wait()
    pltpu.sync_copy(res_v, out_hbm.at[pl.ds(wid * (B // NW), B // NW)])


@functools.partial(
    pl.kernel,
    mesh=plsc.VectorSubcoreMesh(core_axis_name="c", subcore_axis_name="s"),
    out_type=jax.ShapeDtypeStruct((B,), jnp.float32),
    compiler_params=pltpu.CompilerParams(use_tc_tiling_on_sc=False),
    scratch_types=[
        pltpu.VMEM((GR,), jnp.int32),
        pltpu.VMEM((GR,), jnp.int32),
        pltpu.VMEM((GR, ROWI), jnp.int32),
        pltpu.VMEM((GR, ROWI), jnp.int32),
        pltpu.VMEM((B // NW,), jnp.float32),
        pltpu.VMEM((16,), jnp.float32),
        pltpu.SemaphoreType.DMA,
        pltpu.SemaphoreType.DMA,
        pltpu.SemaphoreType.DMA,
        pltpu.SemaphoreType.DMA,
    ],
)
def _ffm_sc(sidx_hbm, bias_hbm, wt_hbm, out_hbm,
            idx_a, idx_b, gbuf_a, gbuf_b, res_v, bias_v,
            sem_a, sem_b, isem_a, isem_b):
    _sc_body(sidx_hbm, bias_hbm, wt_hbm, out_hbm,
             idx_a, idx_b, gbuf_a, gbuf_b, res_v, bias_v,
             sem_a, sem_b, isem_a, isem_b)


def kernel(x, linear_w, bias, ffm_w):
    idx = (x.astype(jnp.int32) + jnp.asarray(_OFFSETS)[None, :])
    sidx = jnp.pad(idx.reshape(NG, GR), ((0, 4), (0, 0)))
    bias16 = jnp.broadcast_to(bias.astype(jnp.float32), (16,))
    linp = jnp.pad(linear_w.astype(jnp.float32), ((0, 0), (0, ED - 1)))
    ffm_i = lax.bitcast_convert_type(
        ffm_w.astype(jnp.bfloat16).reshape(NF, TOT, ED // 2, 2), jnp.int32)
    lin_i = lax.bitcast_convert_type(
        linp.astype(jnp.bfloat16).reshape(TOT, ED // 2, 2), jnp.int32)
    wti = jnp.concatenate(
        [ffm_i.transpose(1, 0, 2), lin_i[:, None, :],
         jnp.zeros((TOT, 1, ED // 2), jnp.int32)], axis=1
    ).reshape(TOT, ROWI)
    return _ffm_sc(sidx, bias16, wti)
